# Initial kernel scaffold; baseline (speedup 1.0000x reference)
import jax
import jax.numpy as jnp
from jax.experimental import pallas as pl


def kernel(x, table):
    raise NotImplementedError("write your pallas kernel here")


# speedup vs baseline: 4.5514x; 4.5514x over previous
import functools
import math

import jax
import jax.numpy as jnp
from jax import lax
from jax.experimental import pallas as pl
from jax.experimental.pallas import tpu as pltpu
from jax.experimental.pallas import tpu_sc as plsc

HID = 32
SCALE = math.sqrt(32.0)

IDX_MINOR = 128
ROWS_PER_CHUNK = 1024
GATHERS_PER_CHUNK = ROWS_PER_CHUNK // IDX_MINOR


def _scale_table(table):
    flat = table.reshape(-1, 128)
    nrows = flat.shape[0]
    br = 2000
    assert nrows % br == 0

    def body(t_ref, o_ref):
        o_ref[...] = t_ref[...] * SCALE

    out = pl.pallas_call(
        body,
        grid=(nrows // br,),
        in_specs=[pl.BlockSpec((br, 128), lambda i: (i, 0))],
        out_specs=pl.BlockSpec((br, 128), lambda i: (i, 0)),
        out_shape=jax.ShapeDtypeStruct((nrows, 128), jnp.float32),
    )(flat)
    return out.reshape(table.shape)


def _sc_gather(idx2d, table, n_total):
    info = plsc.get_sparse_core_info()
    num_workers = info.num_cores * info.num_subcores
    per_worker = n_total // num_workers
    idx_rows_per_worker = per_worker // IDX_MINOR
    chunks = per_worker // ROWS_PER_CHUNK

    mesh = plsc.VectorSubcoreMesh(core_axis_name="c", subcore_axis_name="s")

    @functools.partial(
        pl.kernel,
        mesh=mesh,
        out_type=jax.ShapeDtypeStruct((n_total, HID), jnp.float32),
        compiler_params=pltpu.CompilerParams(use_tc_tiling_on_sc=False),
        scratch_types=[
            pltpu.VMEM((GATHERS_PER_CHUNK, IDX_MINOR), jnp.int32),
            pltpu.VMEM((ROWS_PER_CHUNK, HID), jnp.float32),
            pltpu.SemaphoreType.DMA,
        ],
    )
    def k(idx_hbm, table_hbm, out_hbm, idx_v, rows_v, sem):
        wid = lax.axis_index("s") * info.num_cores + lax.axis_index("c")
        idx_row0 = wid * idx_rows_per_worker
        out_row0 = wid * per_worker

        def body(j, carry):
            pltpu.sync_copy(
                idx_hbm.at[pl.ds(idx_row0 + j * GATHERS_PER_CHUNK,
                                 GATHERS_PER_CHUNK)],
                idx_v)
            copies = [
                pltpu.async_copy(
                    table_hbm.at[idx_v.at[t]],
                    rows_v.at[pl.ds(t * IDX_MINOR, IDX_MINOR)],
                    sem)
                for t in range(GATHERS_PER_CHUNK)
            ]
            for c in copies:
                c.wait()
            pltpu.sync_copy(
                rows_v,
                out_hbm.at[pl.ds(out_row0 + j * ROWS_PER_CHUNK,
                                 ROWS_PER_CHUNK)])
            return carry

        lax.fori_loop(0, chunks, body, 0)

    return k(idx2d, table)


def kernel(x, table):
    b, l = x.shape
    n_total = b * l
    idx2d = x.reshape(n_total // IDX_MINOR, IDX_MINOR).astype(jnp.int32)
    scaled = _scale_table(table)
    out = _sc_gather(idx2d, scaled, n_total)
    return out.reshape(b, l, HID)


# speedup vs baseline: 4.9437x; 1.0862x over previous
import functools
import math

import jax
import jax.numpy as jnp
from jax import lax
from jax.experimental import pallas as pl
from jax.experimental.pallas import tpu as pltpu
from jax.experimental.pallas import tpu_sc as plsc

B = 16384
L = 200
HID = 32
SCALE = math.sqrt(32.0)

IDX_MINOR = 100
CB = 8
ROWS_PER_CHUNK = CB * L
GATHERS_PER_CHUNK = ROWS_PER_CHUNK // IDX_MINOR


def _sc_embed(idx2d, table):
    info = plsc.get_sparse_core_info()
    num_workers = info.num_cores * info.num_subcores
    b_per_w = B // num_workers
    chunks = b_per_w // CB
    idx_rows_per_chunk = ROWS_PER_CHUNK // IDX_MINOR

    mesh = plsc.VectorSubcoreMesh(core_axis_name="c", subcore_axis_name="s")

    @functools.partial(
        pl.kernel,
        mesh=mesh,
        out_type=jax.ShapeDtypeStruct((B, L, HID), jnp.float32),
        compiler_params=pltpu.CompilerParams(use_tc_tiling_on_sc=False),
        scratch_types=[
            pltpu.VMEM((GATHERS_PER_CHUNK, IDX_MINOR), jnp.int32),
            pltpu.VMEM((GATHERS_PER_CHUNK, IDX_MINOR), jnp.int32),
            pltpu.VMEM((ROWS_PER_CHUNK, HID), jnp.float32),
            pltpu.VMEM((ROWS_PER_CHUNK, HID), jnp.float32),
            pltpu.SemaphoreType.DMA,
            pltpu.SemaphoreType.DMA,
            pltpu.SemaphoreType.DMA,
            pltpu.SemaphoreType.DMA,
        ],
    )
    def k(idx_hbm, table_hbm, out_hbm,
          idx_a, idx_b, rows_a, rows_b, gsem_a, gsem_b, ssem_a, ssem_b):
        wid = lax.axis_index("s") * info.num_cores + lax.axis_index("c")
        b0 = wid * b_per_w
        r0 = b0 * (L // IDX_MINOR)

        idx_bufs = (idx_a, idx_b)
        row_bufs = (rows_a, rows_b)
        gsems = (gsem_a, gsem_b)
        ssems = (ssem_a, ssem_b)

        def fire_chunk(c, slot):
            pltpu.sync_copy(
                idx_hbm.at[pl.ds(r0 + c * idx_rows_per_chunk,
                                 idx_rows_per_chunk)],
                idx_bufs[slot])
            for t in range(GATHERS_PER_CHUNK):
                pltpu.make_async_copy(
                    table_hbm.at[idx_bufs[slot].at[t]],
                    row_bufs[slot].at[pl.ds(t * IDX_MINOR, IDX_MINOR)],
                    gsems[slot]).start()

        def wait_gathers(slot):
            for t in range(GATHERS_PER_CHUNK):
                pltpu.make_async_copy(
                    table_hbm.at[idx_bufs[slot].at[t]],
                    row_bufs[slot].at[pl.ds(t * IDX_MINOR, IDX_MINOR)],
                    gsems[slot]).wait()

        def scale_chunk(slot):
            rows = row_bufs[slot]

            @plsc.parallel_loop(0, ROWS_PER_CHUNK, unroll=8)
            def _(r):
                rows[r, pl.ds(0, 16)] = rows[r, pl.ds(0, 16)] * SCALE
                rows[r, pl.ds(16, 16)] = rows[r, pl.ds(16, 16)] * SCALE

        def fire_out(c, slot):
            for bi in range(CB):
                pltpu.make_async_copy(
                    row_bufs[slot].at[pl.ds(bi * L, L)],
                    out_hbm.at[b0 + c * CB + bi],
                    ssems[slot]).start()

        def wait_out(c, slot):
            for bi in range(CB):
                pltpu.make_async_copy(
                    row_bufs[slot].at[pl.ds(bi * L, L)],
                    out_hbm.at[b0 + c * CB + bi],
                    ssems[slot]).wait()

        fire_chunk(0, 0)

        def body(s, carry):
            c0 = 2 * s
            c1 = c0 + 1

            @pl.when(s > 0)
            def _():
                wait_out(c1 - 2, 1)
            fire_chunk(c1, 1)

            wait_gathers(0)
            scale_chunk(0)
            fire_out(c0, 0)

            @pl.when(s < chunks // 2 - 1)
            def _():
                wait_out(c0, 0)
                fire_chunk(c0 + 2, 0)

            wait_gathers(1)
            scale_chunk(1)
            fire_out(c1, 1)
            return carry

        lax.fori_loop(0, chunks // 2, body, 0)
        wait_out(chunks - 2, 0)
        wait_out(chunks - 1, 1)

    return k(idx2d, table)


def kernel(x, table):
    idx2d = x.reshape(B * L // IDX_MINOR, IDX_MINOR).astype(jnp.int32)
    return _sc_embed(idx2d, table)
